# baseline (speedup 1.0000x reference)
import functools

import jax
import jax.numpy as jnp
from jax import lax
from jax.experimental import pallas as pl
from jax.experimental.pallas import tpu as pltpu

R = 6400
CH = 640
SEG_W = 64
BIG = 1 << 30


def _body(x_ref, ids_ref, w1_ref, b1_ref, w2_ref, b2_ref, w3_ref, b3_ref,
          out_ref, acc_ref, cnt_ref, *, nblocks, s_out):
    i = pl.program_id(0)

    @pl.when(i == 0)
    def _init():
        acc_ref[...] = jnp.zeros_like(acc_ref)
        cnt_ref[...] = jnp.zeros_like(cnt_ref)

    x = x_ref[...]
    h = jnp.maximum(
        jnp.dot(x, w1_ref[...], preferred_element_type=jnp.float32)
        + b1_ref[...], 0.0)
    h = jnp.maximum(
        jnp.dot(h, w2_ref[...], preferred_element_type=jnp.float32)
        + b2_ref[...], 0.0)
    y = (jnp.dot(h, w3_ref[...], preferred_element_type=jnp.float32)
         + b3_ref[...])

    ids = ids_ref[0]

    for c in range(R // CH):
        ids_c = ids[:, c * CH:(c + 1) * CH]
        y_c = y[c * CH:(c + 1) * CH, :]
        ids_c_max = jnp.max(ids_c)

        def window(w, ids_c=ids_c, y_c=y_c):
            local = ids_c - w
            iota = lax.broadcasted_iota(jnp.int32, (SEG_W, CH), 0)
            onehot = (iota == local).astype(jnp.float32)
            contrib = jnp.dot(onehot, y_c, preferred_element_type=jnp.float32)
            acc_ref[pl.ds(w, SEG_W), :] += contrib
            cnt1 = jnp.sum(onehot, axis=1, keepdims=True)
            cnt_ref[pl.ds(w, SEG_W), :] += jnp.broadcast_to(cnt1, (SEG_W, 128))
            nxt = jnp.min(jnp.where(ids_c >= w + SEG_W, ids_c, BIG))
            return (nxt // 8) * 8

        w0 = (jnp.min(ids_c) // 8) * 8
        lax.while_loop(lambda w, m=ids_c_max: w <= m, window, w0)

    @pl.when(i == nblocks - 1)
    def _finish():
        a = acc_ref[0:s_out, :]
        c = cnt_ref[0:s_out, :]
        out_ref[...] = jnp.where(c > 0.0, a / jnp.maximum(c, 1.0), 0.0)


def kernel(f_flat, segment_ids, last_hidden, sub_batches, W1, b1, W2, b2, W3, b3):
    n, d = f_flat.shape
    s_out, hdim = last_hidden.shape
    assert n % R == 0 and R % CH == 0
    nblocks = n // R
    s_pad = ((s_out + 7) // 8) * 8 + SEG_W

    ids = segment_ids.astype(jnp.int32).reshape(nblocks, 1, R)

    out = pl.pallas_call(
        functools.partial(_body, nblocks=nblocks, s_out=s_out),
        grid=(nblocks,),
        in_specs=[
            pl.BlockSpec((R, d), lambda i: (i, 0)),
            pl.BlockSpec((1, 1, R), lambda i: (i, 0, 0)),
            pl.BlockSpec(W1.shape, lambda i: (0, 0)),
            pl.BlockSpec((1, W1.shape[1]), lambda i: (0, 0)),
            pl.BlockSpec(W2.shape, lambda i: (0, 0)),
            pl.BlockSpec((1, W2.shape[1]), lambda i: (0, 0)),
            pl.BlockSpec(W3.shape, lambda i: (0, 0)),
            pl.BlockSpec((1, W3.shape[1]), lambda i: (0, 0)),
        ],
        out_specs=pl.BlockSpec((s_out, hdim), lambda i: (0, 0)),
        out_shape=jax.ShapeDtypeStruct((s_out, hdim), jnp.float32),
        scratch_shapes=[
            pltpu.VMEM((s_pad, hdim), jnp.float32),
            pltpu.VMEM((s_pad, hdim), jnp.float32),
        ],
    )(f_flat, ids, W1, b1.reshape(1, -1), W2, b2.reshape(1, -1),
      W3, b3.reshape(1, -1))
    return out


# speedup vs baseline: 2.0953x; 2.0953x over previous
import functools

import jax
import jax.numpy as jnp
from jax import lax
from jax.experimental import pallas as pl
from jax.experimental.pallas import tpu as pltpu

R = 6400
SEG_W = 256
BIG = 1 << 30


def _body(x_ref, ids_ref, w1_ref, b1_ref, w2_ref, b2_ref, w3_ref, b3_ref,
          out_ref, acc_ref, cnt_ref, *, nblocks, s_out):
    i = pl.program_id(0)

    @pl.when(i == 0)
    def _init():
        acc_ref[...] = jnp.zeros_like(acc_ref)
        cnt_ref[...] = jnp.zeros_like(cnt_ref)

    ids = ids_ref[0]
    ids_max = jnp.max(ids)
    w0 = (jnp.min(ids) // 8) * 8

    local0 = ids - w0
    iota = lax.broadcasted_iota(jnp.int32, (SEG_W, R), 0)
    onehot0 = (iota == local0).astype(jnp.float32)
    cnt0 = jnp.sum(onehot0, axis=1, keepdims=True)

    x = x_ref[...]
    h = jnp.maximum(
        jnp.dot(x, w1_ref[...], preferred_element_type=jnp.float32)
        + b1_ref[...], 0.0)
    h = jnp.maximum(
        jnp.dot(h, w2_ref[...], preferred_element_type=jnp.float32)
        + b2_ref[...], 0.0)
    y = (jnp.dot(h, w3_ref[...], preferred_element_type=jnp.float32)
         + b3_ref[...])

    acc_ref[pl.ds(w0, SEG_W), :] += jnp.dot(
        onehot0, y, preferred_element_type=jnp.float32)
    cnt_ref[pl.ds(w0, SEG_W), :] += jnp.broadcast_to(cnt0, (SEG_W, 128))

    def window(w):
        local = ids - w
        onehot = (iota == local).astype(jnp.float32)
        contrib = jnp.dot(onehot, y, preferred_element_type=jnp.float32)
        acc_ref[pl.ds(w, SEG_W), :] += contrib
        cnt1 = jnp.sum(onehot, axis=1, keepdims=True)
        cnt_ref[pl.ds(w, SEG_W), :] += jnp.broadcast_to(cnt1, (SEG_W, 128))
        nxt = jnp.min(jnp.where(ids >= w + SEG_W, ids, BIG))
        return (nxt // 8) * 8

    w1 = (jnp.min(jnp.where(ids >= w0 + SEG_W, ids, BIG)) // 8) * 8
    lax.while_loop(lambda w: w <= ids_max, window, w1)

    @pl.when(i == nblocks - 1)
    def _finish():
        a = acc_ref[0:s_out, :]
        c = cnt_ref[0:s_out, :]
        out_ref[...] = jnp.where(c > 0.0, a / jnp.maximum(c, 1.0), 0.0)


def kernel(f_flat, segment_ids, last_hidden, sub_batches, W1, b1, W2, b2, W3, b3):
    n, d = f_flat.shape
    s_out, hdim = last_hidden.shape
    assert n % R == 0
    nblocks = n // R
    s_pad = ((s_out + 7) // 8) * 8 + SEG_W

    ids = segment_ids.astype(jnp.int32).reshape(nblocks, 1, R)

    out = pl.pallas_call(
        functools.partial(_body, nblocks=nblocks, s_out=s_out),
        grid=(nblocks,),
        in_specs=[
            pl.BlockSpec((R, d), lambda i: (i, 0)),
            pl.BlockSpec((1, 1, R), lambda i: (i, 0, 0)),
            pl.BlockSpec(W1.shape, lambda i: (0, 0)),
            pl.BlockSpec((1, W1.shape[1]), lambda i: (0, 0)),
            pl.BlockSpec(W2.shape, lambda i: (0, 0)),
            pl.BlockSpec((1, W2.shape[1]), lambda i: (0, 0)),
            pl.BlockSpec(W3.shape, lambda i: (0, 0)),
            pl.BlockSpec((1, W3.shape[1]), lambda i: (0, 0)),
        ],
        out_specs=pl.BlockSpec((s_out, hdim), lambda i: (0, 0)),
        out_shape=jax.ShapeDtypeStruct((s_out, hdim), jnp.float32),
        scratch_shapes=[
            pltpu.VMEM((s_pad, hdim), jnp.float32),
            pltpu.VMEM((s_pad, hdim), jnp.float32),
        ],
    )(f_flat, ids, W1, b1.reshape(1, -1), W2, b2.reshape(1, -1),
      W3, b3.reshape(1, -1))
    return out
